# baseline (speedup 1.0000x reference)
import jax
import jax.numpy as jnp
from jax.experimental import pallas as pl
from jax.experimental.pallas import tpu as pltpu

_TABLE = 5


def _stream_body(idx_hbm, out_hbm, buf, sin, sout):
    pltpu.make_async_copy(idx_hbm, buf, sin).start()
    pltpu.make_async_copy(idx_hbm, buf, sin).wait()
    buf[...] = jnp.clip(buf[...], 0, _TABLE - 1)
    pltpu.make_async_copy(buf, out_hbm, sout).start()
    pltpu.make_async_copy(buf, out_hbm, sout).wait()


def kernel(indices, state):
    rows, cols = indices.shape
    out = pl.pallas_call(
        _stream_body,
        in_specs=[pl.BlockSpec(memory_space=pl.ANY)],
        out_specs=pl.BlockSpec(memory_space=pl.ANY),
        out_shape=jax.ShapeDtypeStruct((rows, cols), jnp.int32),
        scratch_shapes=[
            pltpu.VMEM((rows, cols), jnp.int32),
            pltpu.SemaphoreType.DMA,
            pltpu.SemaphoreType.DMA,
        ],
    )(indices)
    return out, state
